# Initial kernel scaffold; baseline (speedup 1.0000x reference)
import functools

import jax
import jax.numpy as jnp
from jax import lax
from jax.experimental import pallas as pl
from jax.experimental.pallas import tpu as pltpu
from jax.experimental.pallas import tpu_sc as plsc

N = 10000
D = 256
E = 160000
HALF = 128
NCORE = 2
NSUB = 16
CHUNK = 128
NB = 79
E_PAD = NSUB * NB * CHUNK
ROWS_ACC = N + 16
RPW = N // NSUB
ZROWS = 125


def _sc_aggregate(x2, src2, dst3):
    mesh = plsc.VectorSubcoreMesh(core_axis_name="c", subcore_axis_name="s")

    @functools.partial(
        pl.kernel,
        mesh=mesh,
        out_type=jax.ShapeDtypeStruct((NCORE, N, HALF), jnp.float32),
        scratch_types=[
            pltpu.VMEM((NB, CHUNK), jnp.int32),
            pltpu.VMEM((NB, CHUNK), jnp.int32),
            pltpu.VMEM((CHUNK, HALF), jnp.float32),
            pltpu.VMEM((ZROWS, HALF), jnp.float32),
            pltpu.VMEM_SHARED((ROWS_ACC, HALF), jnp.float32),
            pltpu.SemaphoreType.DMA,
        ],
    )
    def k(src2_hbm, dst3_hbm, x2_hbm, out_hbm, src_v, dst_v, rows_v, zbuf, acc, sem):
        c = lax.axis_index("c")
        s = lax.axis_index("s")

        pltpu.sync_copy(src2_hbm.at[c, s], src_v)
        pltpu.sync_copy(dst3_hbm.at[s], dst_v)

        def zrow(i, carry):
            def zcol(j, carry2):
                zbuf[i, pl.ds(j * 16, 16)] = jnp.zeros((16,), jnp.float32)
                return carry2
            return lax.fori_loop(0, HALF // 16, zcol, carry)
        lax.fori_loop(0, ZROWS, zrow, 0)
        for t in range(RPW // ZROWS):
            pltpu.sync_copy(zbuf, acc.at[pl.ds(s * RPW + t * ZROWS, ZROWS), :])
        plsc.subcore_barrier()

        def body(j, carry):
            pltpu.async_copy(x2_hbm.at[src_v.at[j]], rows_v, sem).wait()
            pltpu.sync_copy(rows_v, acc.at[dst_v.at[j]], add=True)
            return carry
        lax.fori_loop(0, NB, body, 0)
        plsc.subcore_barrier()

        pltpu.sync_copy(acc.at[pl.ds(s * RPW, RPW), :],
                        out_hbm.at[c, pl.ds(s * RPW, RPW)])

    return k(src2, dst3, x2)


def _tc_dense(x, aggr2, W, b, eps):
    R = 1000

    def body(eps_ref, x_ref, a_ref, w_ref, b_ref, o_ref):
        e1 = 1.0 + eps_ref[0, 0]
        w = w_ref[...]
        h0 = e1 * x_ref[:, :HALF] + a_ref[0]
        h1 = e1 * x_ref[:, HALF:] + a_ref[1]
        acc = lax.dot_general(h0, w[:, :HALF], (((1,), (1,)), ((), ())),
                              preferred_element_type=jnp.float32)
        acc = acc + lax.dot_general(h1, w[:, HALF:], (((1,), (1,)), ((), ())),
                                    preferred_element_type=jnp.float32)
        o_ref[...] = jnp.maximum(acc + b_ref[...], 0.0)

    return pl.pallas_call(
        body,
        grid=(N // R,),
        in_specs=[
            pl.BlockSpec(memory_space=pltpu.SMEM),
            pl.BlockSpec((R, D), lambda i: (i, 0)),
            pl.BlockSpec((NCORE, R, HALF), lambda i: (0, i, 0)),
            pl.BlockSpec((D, D), lambda i: (0, 0)),
            pl.BlockSpec((1, D), lambda i: (0, 0)),
        ],
        out_specs=pl.BlockSpec((R, D), lambda i: (i, 0)),
        out_shape=jax.ShapeDtypeStruct((N, D), jnp.float32),
    )(eps.reshape(1, 1).astype(jnp.float32), x, aggr2, W, b.reshape(1, D))


def kernel(x, edge_index, W, b, eps):
    src = edge_index[0]
    dst = edge_index[1]
    pad = E_PAD - E
    pad_src = jnp.arange(pad, dtype=jnp.int32) % jnp.int32(N)
    pad_dst = jnp.int32(N) + jnp.arange(pad, dtype=jnp.int32) % jnp.int32(16)
    srcp = jnp.concatenate([src, pad_src])
    dstp = jnp.concatenate([dst, pad_dst])
    base = srcp * 2
    src2 = jnp.stack([base, base + 1]).reshape(NCORE, NSUB, NB, CHUNK)
    dst3 = dstp.reshape(NSUB, NB, CHUNK)
    x2 = x.reshape(2 * N, HALF)
    aggr2 = _sc_aggregate(x2, src2, dst3)
    return _tc_dense(x, aggr2, W, b, eps)


# speedup vs baseline: 5.7294x; 5.7294x over previous
import functools

import jax
import jax.numpy as jnp
from jax import lax
from jax.experimental import pallas as pl
from jax.experimental.pallas import tpu as pltpu
from jax.experimental.pallas import tpu_sc as plsc

N = 10000
D = 256
E = 160000
HALF = 128
NCORE = 2
NSUB = 16
CHUNK = 128
NB = 79
E_PAD = NSUB * NB * CHUNK
ROWS_ACC = N + 16
RPW = 624
TAIL = N - NSUB * RPW


def _sc_aggregate(x2, src2, dst3):
    mesh = plsc.VectorSubcoreMesh(core_axis_name="c", subcore_axis_name="s")

    @functools.partial(
        pl.kernel,
        mesh=mesh,
        out_type=jax.ShapeDtypeStruct((NCORE, N, HALF), jnp.float32),
        scratch_types=[
            pltpu.VMEM((NB, CHUNK), jnp.int32),
            pltpu.VMEM((NB, CHUNK), jnp.int32),
            pltpu.VMEM((CHUNK, HALF), jnp.float32),
            pltpu.VMEM_SHARED((ROWS_ACC, HALF), jnp.float32),
            pltpu.SemaphoreType.DMA,
        ],
    )
    def k(src2_hbm, dst3_hbm, x2_hbm, out_hbm, src_v, dst_v, rows_v, acc, sem):
        c = lax.axis_index("c")
        s = lax.axis_index("s")

        pltpu.sync_copy(src2_hbm.at[c, s], src_v)
        pltpu.sync_copy(dst3_hbm.at[s], dst_v)

        def zrow(i, carry):
            def zcol(j, carry2):
                rows_v[i, pl.ds(j * 16, 16)] = jnp.zeros((16,), jnp.float32)
                return carry2
            return lax.fori_loop(0, HALF // 16, zcol, carry)
        lax.fori_loop(0, CHUNK, zrow, 0)
        for t in range(RPW // CHUNK):
            pltpu.sync_copy(rows_v, acc.at[pl.ds(s * RPW + t * CHUNK, CHUNK), :])
        rem = RPW - (RPW // CHUNK) * CHUNK
        pltpu.sync_copy(rows_v.at[pl.ds(0, rem), :],
                        acc.at[pl.ds(s * RPW + RPW - rem, rem), :])

        @pl.when(s == NSUB - 1)
        def _zero_tail():
            pltpu.sync_copy(rows_v.at[pl.ds(0, TAIL), :],
                            acc.at[pl.ds(NSUB * RPW, TAIL), :])
        plsc.subcore_barrier()

        def body(j, carry):
            pltpu.async_copy(x2_hbm.at[src_v.at[j]], rows_v, sem).wait()
            pltpu.sync_copy(rows_v, acc.at[dst_v.at[j]], add=True)
            return carry
        lax.fori_loop(0, NB, body, 0)
        plsc.subcore_barrier()

        pltpu.sync_copy(acc.at[pl.ds(s * RPW, RPW), :],
                        out_hbm.at[c, pl.ds(s * RPW, RPW)])

        @pl.when(s == NSUB - 1)
        def _write_tail():
            pltpu.sync_copy(acc.at[pl.ds(NSUB * RPW, TAIL), :],
                            out_hbm.at[c, pl.ds(NSUB * RPW, TAIL)])

    return k(src2, dst3, x2)


def _tc_dense(x, aggr2, W, b, eps):
    R = 1000

    def body(eps_ref, x_ref, a_ref, w_ref, b_ref, o_ref):
        e1 = 1.0 + eps_ref[0, 0]
        w = w_ref[...]
        h0 = e1 * x_ref[:, :HALF] + a_ref[0]
        h1 = e1 * x_ref[:, HALF:] + a_ref[1]
        acc = lax.dot_general(h0, w[:, :HALF], (((1,), (1,)), ((), ())),
                              preferred_element_type=jnp.float32)
        acc = acc + lax.dot_general(h1, w[:, HALF:], (((1,), (1,)), ((), ())),
                                    preferred_element_type=jnp.float32)
        o_ref[...] = jnp.maximum(acc + b_ref[...], 0.0)

    return pl.pallas_call(
        body,
        grid=(N // R,),
        in_specs=[
            pl.BlockSpec(memory_space=pltpu.SMEM),
            pl.BlockSpec((R, D), lambda i: (i, 0)),
            pl.BlockSpec((NCORE, R, HALF), lambda i: (0, i, 0)),
            pl.BlockSpec((D, D), lambda i: (0, 0)),
            pl.BlockSpec((1, D), lambda i: (0, 0)),
        ],
        out_specs=pl.BlockSpec((R, D), lambda i: (i, 0)),
        out_shape=jax.ShapeDtypeStruct((N, D), jnp.float32),
    )(eps.reshape(1, 1).astype(jnp.float32), x, aggr2, W, b.reshape(1, D))


def kernel(x, edge_index, W, b, eps):
    src = edge_index[0]
    dst = edge_index[1]
    pad = E_PAD - E
    pad_src = jnp.arange(pad, dtype=jnp.int32) % jnp.int32(N)
    pad_dst = jnp.int32(N) + jnp.arange(pad, dtype=jnp.int32) % jnp.int32(16)
    srcp = jnp.concatenate([src, pad_src])
    dstp = jnp.concatenate([dst, pad_dst])
    base = srcp * 2
    src2 = jnp.stack([base, base + 1]).reshape(NCORE, NSUB, NB, CHUNK)
    dst3 = dstp.reshape(NSUB, NB, CHUNK)
    x2 = x.reshape(2 * N, HALF)
    aggr2 = _sc_aggregate(x2, src2, dst3)
    return _tc_dense(x, aggr2, W, b, eps)


# speedup vs baseline: 8.6330x; 1.5068x over previous
import functools

import jax
import jax.numpy as jnp
from jax import lax
from jax.experimental import pallas as pl
from jax.experimental.pallas import tpu as pltpu
from jax.experimental.pallas import tpu_sc as plsc

N = 10000
D = 256
E = 160000
HALF = 128
NCORE = 2
NSUB = 16
CHUNK = 128
NB = 80
E_PAD = NSUB * NB * CHUNK
ROWS_ACC = N + 16
RPW = 624
TAIL = N - NSUB * RPW


def _sc_aggregate(x2, packed):
    mesh = plsc.VectorSubcoreMesh(core_axis_name="c", subcore_axis_name="s")

    @functools.partial(
        pl.kernel,
        mesh=mesh,
        out_type=jax.ShapeDtypeStruct((NCORE, N, HALF), jnp.float32),
        scratch_types=[
            pltpu.VMEM((NB, CHUNK), jnp.int32),
            pltpu.VMEM((8, CHUNK), jnp.int32),
            pltpu.VMEM((2, CHUNK, HALF), jnp.float32),
            pltpu.VMEM_SHARED((ROWS_ACC, HALF), jnp.float32),
            pltpu.SemaphoreType.DMA,
            pltpu.SemaphoreType.DMA,
        ],
    )
    def k(packed_hbm, x2_hbm, out_hbm, pk_v, su, rows_v, acc, sem0, sem1):
        c = lax.axis_index("c")
        s = lax.axis_index("s")

        pltpu.sync_copy(packed_hbm.at[c, s], pk_v)

        def zrow(i, carry):
            def zcol(j, carry2):
                rows_v[0, i, pl.ds(j * 16, 16)] = jnp.zeros((16,), jnp.float32)
                return carry2
            return lax.fori_loop(0, HALF // 16, zcol, carry)
        lax.fori_loop(0, CHUNK, zrow, 0)
        zslab = rows_v.at[0]
        for t in range(RPW // CHUNK):
            pltpu.sync_copy(zslab, acc.at[pl.ds(s * RPW + t * CHUNK, CHUNK), :])
        rem = RPW - (RPW // CHUNK) * CHUNK
        if rem:
            pltpu.sync_copy(zslab.at[pl.ds(0, rem), :],
                            acc.at[pl.ds(s * RPW + RPW - rem, rem), :])

        @pl.when(s == NSUB - 1)
        def _zero_tail():
            pltpu.sync_copy(zslab.at[pl.ds(0, TAIL), :],
                            acc.at[pl.ds(NSUB * RPW, TAIL), :])
        plsc.subcore_barrier()

        sems = (sem0, sem1)

        def unpack(j, buf):
            for t in range(CHUNK // 16):
                p = pk_v[j, pl.ds(t * 16, 16)]
                su[buf, pl.ds(t * 16, 16)] = jnp.right_shift(p, 14)
                su[2 + buf, pl.ds(t * 16, 16)] = jnp.bitwise_and(p, 16383)

        def start_gather(j, buf):
            unpack(j, buf)
            pltpu.async_copy(x2_hbm.at[su.at[buf]], rows_v.at[buf], sems[buf])

        def finish_chunk(buf):
            pltpu.make_async_copy(x2_hbm.at[su.at[buf]], rows_v.at[buf],
                                  sems[buf]).wait()
            pltpu.sync_copy(rows_v.at[buf], acc.at[su.at[2 + buf]], add=True)

        start_gather(0, 0)

        def body(t, carry):
            j = 2 * t
            for bb in range(2):
                jj = j + bb

                @pl.when(jj + 1 < NB)
                def _next():
                    start_gather(jj + 1, 1 - bb)

                finish_chunk(bb)
            return carry
        lax.fori_loop(0, NB // 2, body, 0)
        plsc.subcore_barrier()

        pltpu.sync_copy(acc.at[pl.ds(s * RPW, RPW), :],
                        out_hbm.at[c, pl.ds(s * RPW, RPW)])

        @pl.when(s == NSUB - 1)
        def _write_tail():
            pltpu.sync_copy(acc.at[pl.ds(NSUB * RPW, TAIL), :],
                            out_hbm.at[c, pl.ds(NSUB * RPW, TAIL)])

    return k(packed, x2)


def _tc_dense(x, aggr2, W, b, eps):
    R = 1000

    def body(eps_ref, x_ref, a_ref, w_ref, b_ref, o_ref):
        e1 = 1.0 + eps_ref[0, 0]
        w = w_ref[...]
        h0 = e1 * x_ref[:, :HALF] + a_ref[0]
        h1 = e1 * x_ref[:, HALF:] + a_ref[1]
        acc = lax.dot_general(h0, w[:, :HALF], (((1,), (1,)), ((), ())),
                              preferred_element_type=jnp.float32)
        acc = acc + lax.dot_general(h1, w[:, HALF:], (((1,), (1,)), ((), ())),
                                    preferred_element_type=jnp.float32)
        o_ref[...] = jnp.maximum(acc + b_ref[...], 0.0)

    return pl.pallas_call(
        body,
        grid=(N // R,),
        in_specs=[
            pl.BlockSpec(memory_space=pltpu.SMEM),
            pl.BlockSpec((R, D), lambda i: (i, 0)),
            pl.BlockSpec((NCORE, R, HALF), lambda i: (0, i, 0)),
            pl.BlockSpec((D, D), lambda i: (0, 0)),
            pl.BlockSpec((1, D), lambda i: (0, 0)),
        ],
        out_specs=pl.BlockSpec((R, D), lambda i: (i, 0)),
        out_shape=jax.ShapeDtypeStruct((N, D), jnp.float32),
    )(eps.reshape(1, 1).astype(jnp.float32), x, aggr2, W, b.reshape(1, D))


def kernel(x, edge_index, W, b, eps):
    src = edge_index[0]
    dst = edge_index[1]
    pad = E_PAD - E
    pad_src = jnp.arange(pad, dtype=jnp.int32) % jnp.int32(N)
    pad_dst = jnp.int32(N) + jnp.arange(pad, dtype=jnp.int32) % jnp.int32(16)
    srcp = jnp.concatenate([src, pad_src])
    dstp = jnp.concatenate([dst, pad_dst])
    p0 = srcp * 32768 + dstp
    p1 = p0 + 16384
    packed = jnp.stack([p0, p1]).reshape(NCORE, NSUB, NB, CHUNK)
    x2 = x.reshape(2 * N, HALF)
    aggr2 = _sc_aggregate(x2, packed)
    return _tc_dense(x, aggr2, W, b, eps)


# speedup vs baseline: 8.9003x; 1.0310x over previous
import functools

import jax
import jax.numpy as jnp
from jax import lax
from jax.experimental import pallas as pl
from jax.experimental.pallas import tpu as pltpu
from jax.experimental.pallas import tpu_sc as plsc

N = 10000
D = 256
E = 160000
HALF = 128
NCORE = 2
NSUB = 16
CHUNK = 128
NB = 80
E_PAD = NSUB * NB * CHUNK
ROWS_ACC = N + 16
RPW = 624
TAIL = N - NSUB * RPW


def _sc_aggregate(x, packed):
    mesh = plsc.VectorSubcoreMesh(core_axis_name="c", subcore_axis_name="s")

    @functools.partial(
        pl.kernel,
        mesh=mesh,
        out_type=jax.ShapeDtypeStruct((NCORE, N, HALF), jnp.float32),
        scratch_types=[
            pltpu.VMEM((NB, CHUNK), jnp.int32),
            pltpu.VMEM((8, CHUNK), jnp.int32),
            pltpu.VMEM((2, CHUNK, HALF), jnp.float32),
            pltpu.VMEM_SHARED((ROWS_ACC, HALF), jnp.float32),
            pltpu.SemaphoreType.DMA,
            pltpu.SemaphoreType.DMA,
        ],
    )
    def k(packed_hbm, x_hbm, out_hbm, pk_v, su, rows_v, acc, sem0, sem1):
        c = lax.axis_index("c")
        s = lax.axis_index("s")
        xh = x_hbm.at[:, pl.ds(c * HALF, HALF)]

        pltpu.sync_copy(packed_hbm.at[s], pk_v)

        def zrow(i, carry):
            def zcol(j, carry2):
                rows_v[0, i, pl.ds(j * 16, 16)] = jnp.zeros((16,), jnp.float32)
                return carry2
            return lax.fori_loop(0, HALF // 16, zcol, carry)
        lax.fori_loop(0, CHUNK, zrow, 0)
        zslab = rows_v.at[0]
        for t in range(RPW // CHUNK):
            pltpu.sync_copy(zslab, acc.at[pl.ds(s * RPW + t * CHUNK, CHUNK), :])
        rem = RPW - (RPW // CHUNK) * CHUNK
        if rem:
            pltpu.sync_copy(zslab.at[pl.ds(0, rem), :],
                            acc.at[pl.ds(s * RPW + RPW - rem, rem), :])

        @pl.when(s == NSUB - 1)
        def _zero_tail():
            pltpu.sync_copy(zslab.at[pl.ds(0, TAIL), :],
                            acc.at[pl.ds(NSUB * RPW, TAIL), :])
        plsc.subcore_barrier()

        sems = (sem0, sem1)

        def unpack(j, buf):
            for t in range(CHUNK // 16):
                p = pk_v[j, pl.ds(t * 16, 16)]
                su[buf, pl.ds(t * 16, 16)] = jnp.right_shift(p, 14)
                su[2 + buf, pl.ds(t * 16, 16)] = jnp.bitwise_and(p, 16383)

        def start_gather(j, buf):
            unpack(j, buf)
            pltpu.async_copy(xh.at[su.at[buf]], rows_v.at[buf], sems[buf])

        def finish_chunk(buf):
            pltpu.make_async_copy(xh.at[su.at[buf]], rows_v.at[buf],
                                  sems[buf]).wait()
            pltpu.sync_copy(rows_v.at[buf], acc.at[su.at[2 + buf]], add=True)

        start_gather(0, 0)

        def body(t, carry):
            j = 2 * t
            for bb in range(2):
                jj = j + bb

                @pl.when(jj + 1 < NB)
                def _next():
                    start_gather(jj + 1, 1 - bb)

                finish_chunk(bb)
            return carry
        lax.fori_loop(0, NB // 2, body, 0)
        plsc.subcore_barrier()

        pltpu.sync_copy(acc.at[pl.ds(s * RPW, RPW), :],
                        out_hbm.at[c, pl.ds(s * RPW, RPW)])

        @pl.when(s == NSUB - 1)
        def _write_tail():
            pltpu.sync_copy(acc.at[pl.ds(NSUB * RPW, TAIL), :],
                            out_hbm.at[c, pl.ds(NSUB * RPW, TAIL)])

    return k(packed, x)


def _tc_dense(x, aggr2, W, b, eps):
    R = 1000

    def body(eps_ref, x_ref, a_ref, w_ref, b_ref, o_ref):
        e1 = 1.0 + eps_ref[0, 0]
        w = w_ref[...]
        h0 = e1 * x_ref[:, :HALF] + a_ref[0]
        h1 = e1 * x_ref[:, HALF:] + a_ref[1]
        acc = lax.dot_general(h0, w[:, :HALF], (((1,), (1,)), ((), ())),
                              preferred_element_type=jnp.float32)
        acc = acc + lax.dot_general(h1, w[:, HALF:], (((1,), (1,)), ((), ())),
                                    preferred_element_type=jnp.float32)
        o_ref[...] = jnp.maximum(acc + b_ref[...], 0.0)

    return pl.pallas_call(
        body,
        grid=(N // R,),
        in_specs=[
            pl.BlockSpec(memory_space=pltpu.SMEM),
            pl.BlockSpec((R, D), lambda i: (i, 0)),
            pl.BlockSpec((NCORE, R, HALF), lambda i: (0, i, 0)),
            pl.BlockSpec((D, D), lambda i: (0, 0)),
            pl.BlockSpec((1, D), lambda i: (0, 0)),
        ],
        out_specs=pl.BlockSpec((R, D), lambda i: (i, 0)),
        out_shape=jax.ShapeDtypeStruct((N, D), jnp.float32),
    )(eps.reshape(1, 1).astype(jnp.float32), x, aggr2, W, b.reshape(1, D))


def kernel(x, edge_index, W, b, eps):
    src = edge_index[0]
    dst = edge_index[1]
    pad = E_PAD - E
    pad_src = jnp.arange(pad, dtype=jnp.int32) % jnp.int32(N)
    pad_dst = jnp.int32(N) + jnp.arange(pad, dtype=jnp.int32) % jnp.int32(16)
    srcp = jnp.concatenate([src, pad_src])
    dstp = jnp.concatenate([dst, pad_dst])
    packed = (srcp * 16384 + dstp).reshape(NSUB, NB, CHUNK)
    aggr2 = _sc_aggregate(x, packed)
    return _tc_dense(x, aggr2, W, b, eps)
